# Initial kernel scaffold; baseline (speedup 1.0000x reference)
import jax
import jax.numpy as jnp
from jax.experimental import pallas as pl


def kernel(FSR_Mat, labels, class_smi_UB, class_smi_LB):
    raise NotImplementedError("write your pallas kernel here")


# speedup vs baseline: 1.2728x; 1.2728x over previous
import functools

import jax
import jax.numpy as jnp
from jax import lax
from jax.experimental import pallas as pl
from jax.experimental.pallas import tpu as pltpu
from jax.experimental.pallas import tpu_sc as plsc

LANES = 16
NCORES = 2
NSUB = 16
NWORKERS = NCORES * NSUB
CPAD = 1024


def _sc_gather_cols(ub_pad, lb_pad, labels):
    B = labels.shape[0]
    rows_per_w = CPAD // NWORKERS
    mesh = plsc.VectorSubcoreMesh(core_axis_name="c", subcore_axis_name="s")

    @functools.partial(
        pl.kernel,
        out_type=(
            jax.ShapeDtypeStruct((CPAD, B), jnp.float32),
            jax.ShapeDtypeStruct((CPAD, B), jnp.float32),
        ),
        mesh=mesh,
        scratch_types=[
            pltpu.VMEM((B,), jnp.int32),
            pltpu.VMEM((rows_per_w * CPAD,), jnp.float32),
            pltpu.VMEM((rows_per_w * CPAD,), jnp.float32),
            pltpu.VMEM((B,), jnp.float32),
            pltpu.VMEM((B,), jnp.float32),
            pltpu.VMEM((B,), jnp.float32),
            pltpu.VMEM((B,), jnp.float32),
            pltpu.SemaphoreType.DMA,
            pltpu.SemaphoreType.DMA,
        ],
        compiler_params=pltpu.CompilerParams(needs_layout_passes=False),
    )
    def k(ub_hbm, lb_hbm, lbl_hbm, out_ub, out_lb,
          lbl_v, ubr, lbr, oub0, oub1, olb0, olb1, sem_in, sem_out):
        wid = lax.axis_index("s") * NCORES + lax.axis_index("c")
        start = wid * rows_per_w
        flat_start = start * CPAD
        pltpu.sync_copy(lbl_hbm, lbl_v)
        cin_u = pltpu.async_copy(
            ub_hbm.at[pl.ds(flat_start, rows_per_w * CPAD)], ubr, sem_in)
        cin_l = pltpu.async_copy(
            lb_hbm.at[pl.ds(flat_start, rows_per_w * CPAD)], lbr, sem_in)
        cin_u.wait()
        cin_l.wait()

        obufs = ((oub0, olb0), (oub1, olb1))
        pending = []
        for r in range(rows_per_w):
            ob_u, ob_l = obufs[r % 2]
            if r >= 2:
                pending[2 * (r - 2)].wait()
                pending[2 * (r - 2) + 1].wait()
            row_base = jnp.full((LANES,), r * CPAD, jnp.int32)

            def jbody(j, _, ob_u=ob_u, ob_l=ob_l, row_base=row_base):
                off = pl.multiple_of(j * LANES, LANES)
                idx = lbl_v[pl.ds(off, LANES)] + row_base
                ob_u[pl.ds(off, LANES)] = plsc.load_gather(ubr, [idx])
                ob_l[pl.ds(off, LANES)] = plsc.load_gather(lbr, [idx])
                return 0

            lax.fori_loop(0, B // LANES, jbody, 0)
            pending.append(pltpu.async_copy(ob_u, out_ub.at[start + r], sem_out))
            pending.append(pltpu.async_copy(ob_l, out_lb.at[start + r], sem_out))
        for cp in pending[2 * (rows_per_w - 2):]:
            cp.wait()

    return k(ub_pad, lb_pad, labels)


def _tc_loss(labels, f3, ub3, lb3):
    n_rows = f3.shape[0]
    block_rows = 256
    grid = (n_rows // block_rows,)

    def body(lbl_ref, f_ref, ub_hbm, lb_hbm, out_ref, ub_v, lb_v, acc_ref, sem):
        i = pl.program_id(0)

        @pl.when(i == 0)
        def _():
            cu = pltpu.make_async_copy(ub_hbm, ub_v, sem)
            cu.start()
            cu.wait()
            cl = pltpu.make_async_copy(lb_hbm, lb_v, sem)
            cl.start()
            cl.wait()
            acc_ref[...] = jnp.zeros_like(acc_ref)

        base = i * block_rows

        def row(r, acc):
            l = lbl_ref[base + r]
            f = f_ref[r]
            u = ub_v[l]
            lo = lb_v[l]
            return acc + jnp.maximum(lo - f, 0.0) + jnp.maximum(f - u, 0.0)

        acc_ref[...] = lax.fori_loop(0, block_rows, row, acc_ref[...])

        @pl.when(i == grid[0] - 1)
        def _():
            out_ref[0] = jnp.sum(acc_ref[...]) / (float(n_rows) * float(n_rows))

    grid_spec = pltpu.PrefetchScalarGridSpec(
        num_scalar_prefetch=1,
        grid=grid,
        in_specs=[
            pl.BlockSpec((block_rows, 32, 128), lambda i, lbl: (i, 0, 0)),
            pl.BlockSpec(memory_space=pltpu.MemorySpace.HBM),
            pl.BlockSpec(memory_space=pltpu.MemorySpace.HBM),
        ],
        out_specs=pl.BlockSpec(memory_space=pltpu.MemorySpace.SMEM),
        scratch_shapes=[
            pltpu.VMEM((CPAD, 32, 128), jnp.float32),
            pltpu.VMEM((CPAD, 32, 128), jnp.float32),
            pltpu.VMEM((32, 128), jnp.float32),
            pltpu.SemaphoreType.DMA,
        ],
    )
    return pl.pallas_call(
        body,
        grid_spec=grid_spec,
        out_shape=jax.ShapeDtypeStruct((1,), jnp.float32),
    )(labels, f3, ub3, lb3)


def kernel(FSR_Mat, labels, class_smi_UB, class_smi_LB):
    n = class_smi_UB.shape[0]
    pad = ((0, CPAD - n), (0, CPAD - n))
    ub_pad = jnp.pad(class_smi_UB, pad)
    lb_pad = jnp.pad(class_smi_LB, pad)
    t_ub, t_lb = _sc_gather_cols(ub_pad.reshape(-1), lb_pad.reshape(-1), labels)
    b = FSR_Mat.shape[0]
    f3 = FSR_Mat.reshape(b, 32, 128)
    out = _tc_loss(labels, f3,
                   t_ub.reshape(CPAD, 32, 128), t_lb.reshape(CPAD, 32, 128))
    return out[0]


# speedup vs baseline: 1.4782x; 1.1614x over previous
import functools

import jax
import jax.numpy as jnp
from jax import lax
from jax.experimental import pallas as pl
from jax.experimental.pallas import tpu as pltpu
from jax.experimental.pallas import tpu_sc as plsc

LANES = 16
NCORES = 2
NSUB = 16
NWORKERS = NCORES * NSUB
NCLS = 1000
WROWS = 32


def _sc_gather_cols(ub_flat, lb_flat, labels):
    B = labels.shape[0]
    mesh = plsc.VectorSubcoreMesh(core_axis_name="c", subcore_axis_name="s")

    @functools.partial(
        pl.kernel,
        out_type=(
            jax.ShapeDtypeStruct((NCLS, B // 128, 128), jnp.float32),
            jax.ShapeDtypeStruct((NCLS, B // 128, 128), jnp.float32),
        ),
        mesh=mesh,
        scratch_types=[
            pltpu.VMEM((B,), jnp.int32),
            pltpu.VMEM((WROWS * NCLS,), jnp.float32),
            pltpu.VMEM((WROWS * NCLS,), jnp.float32),
            pltpu.VMEM((B // 128, 128), jnp.float32),
            pltpu.VMEM((B // 128, 128), jnp.float32),
            pltpu.VMEM((B // 128, 128), jnp.float32),
            pltpu.VMEM((B // 128, 128), jnp.float32),
            pltpu.SemaphoreType.DMA,
            pltpu.SemaphoreType.DMA,
        ],
        compiler_params=pltpu.CompilerParams(needs_layout_passes=False),
    )
    def k(ub_hbm, lb_hbm, lbl_hbm, out_ub, out_lb,
          lbl_v, ubr, lbr, oub0, oub1, olb0, olb1, sem_in, sem_out):
        wid = lax.axis_index("s") * NCORES + lax.axis_index("c")
        start = lax.shift_right_logical(wid * 125, 2)
        flat_start = start * NCLS
        pltpu.sync_copy(lbl_hbm, lbl_v)
        cin_u = pltpu.async_copy(
            ub_hbm.at[pl.ds(flat_start, WROWS * NCLS)], ubr, sem_in)
        cin_l = pltpu.async_copy(
            lb_hbm.at[pl.ds(flat_start, WROWS * NCLS)], lbr, sem_in)
        cin_u.wait()
        cin_l.wait()

        obufs = ((oub0, olb0), (oub1, olb1))
        pending = []
        nsub = B // 128
        for r in range(WROWS):
            ob_u, ob_l = obufs[r % 2]
            if r >= 2:
                pending[2 * (r - 2)].wait()
                pending[2 * (r - 2) + 1].wait()
            row_base = jnp.full((LANES,), r * NCLS, jnp.int32)

            def sbody(s, _, ob_u=ob_u, ob_l=ob_l, row_base=row_base):
                soff = pl.multiple_of(s * 128, 128)
                for kk in range(8):
                    idx = lbl_v[pl.ds(soff + kk * LANES, LANES)] + row_base
                    ob_u[s, pl.ds(kk * LANES, LANES)] = plsc.load_gather(
                        ubr, [idx])
                    ob_l[s, pl.ds(kk * LANES, LANES)] = plsc.load_gather(
                        lbr, [idx])
                return 0

            lax.fori_loop(0, nsub, sbody, 0)
            pending.append(pltpu.async_copy(ob_u, out_ub.at[start + r], sem_out))
            pending.append(pltpu.async_copy(ob_l, out_lb.at[start + r], sem_out))
        for cp in pending[2 * (WROWS - 2):]:
            cp.wait()

    return k(ub_flat, lb_flat, labels)


def _tc_loss(labels, f3, ub3, lb3):
    n_rows = f3.shape[0]
    block_rows = 256
    grid = (n_rows // block_rows,)

    def body(lbl_ref, f_ref, ub_hbm, lb_hbm, out_ref, ub_v, lb_v, acc_ref, sem):
        i = pl.program_id(0)

        @pl.when(i == 0)
        def _():
            cu = pltpu.make_async_copy(ub_hbm, ub_v, sem)
            cu.start()
            cu.wait()
            cl = pltpu.make_async_copy(lb_hbm, lb_v, sem)
            cl.start()
            cl.wait()
            acc_ref[...] = jnp.zeros_like(acc_ref)

        base = i * block_rows

        def row(r, acc):
            l = lbl_ref[base + r]
            f = f_ref[r]
            u = ub_v[l]
            lo = lb_v[l]
            return acc + jnp.maximum(lo - f, 0.0) + jnp.maximum(f - u, 0.0)

        acc_ref[...] = lax.fori_loop(0, block_rows, row, acc_ref[...])

        @pl.when(i == grid[0] - 1)
        def _():
            out_ref[0] = jnp.sum(acc_ref[...]) / (float(n_rows) * float(n_rows))

    grid_spec = pltpu.PrefetchScalarGridSpec(
        num_scalar_prefetch=1,
        grid=grid,
        in_specs=[
            pl.BlockSpec((block_rows, 32, 128), lambda i, lbl: (i, 0, 0)),
            pl.BlockSpec(memory_space=pltpu.MemorySpace.HBM),
            pl.BlockSpec(memory_space=pltpu.MemorySpace.HBM),
        ],
        out_specs=pl.BlockSpec(memory_space=pltpu.MemorySpace.SMEM),
        scratch_shapes=[
            pltpu.VMEM((NCLS, 32, 128), jnp.float32),
            pltpu.VMEM((NCLS, 32, 128), jnp.float32),
            pltpu.VMEM((32, 128), jnp.float32),
            pltpu.SemaphoreType.DMA,
        ],
    )
    return pl.pallas_call(
        body,
        grid_spec=grid_spec,
        out_shape=jax.ShapeDtypeStruct((1,), jnp.float32),
    )(labels, f3, ub3, lb3)


def kernel(FSR_Mat, labels, class_smi_UB, class_smi_LB):
    t_ub, t_lb = _sc_gather_cols(
        class_smi_UB.reshape(-1), class_smi_LB.reshape(-1), labels)
    b = FSR_Mat.shape[0]
    f3 = FSR_Mat.reshape(b, 32, 128)
    return _tc_loss(labels, f3, t_ub, t_lb)[0]


# speedup vs baseline: 2.0925x; 1.4155x over previous
import functools

import jax
import jax.numpy as jnp
from jax import lax
from jax.experimental import pallas as pl
from jax.experimental.pallas import tpu as pltpu
from jax.experimental.pallas import tpu_sc as plsc

LANES = 16
NCORES = 2
NSUB = 16
NWORKERS = NCORES * NSUB
NCLS = 1000
WROWS = 32


def _sc_gather_cols(ub_flat, lb_flat, labels):
    B = labels.shape[0]
    mesh = plsc.VectorSubcoreMesh(core_axis_name="c", subcore_axis_name="s")

    @functools.partial(
        pl.kernel,
        out_type=(
            jax.ShapeDtypeStruct((NCLS, B // 128, 128), jnp.float32),
            jax.ShapeDtypeStruct((NCLS, B // 128, 128), jnp.float32),
        ),
        mesh=mesh,
        scratch_types=[
            pltpu.VMEM((B,), jnp.int32),
            pltpu.VMEM((WROWS * NCLS,), jnp.float32),
            pltpu.VMEM((WROWS * NCLS,), jnp.float32),
            pltpu.VMEM((B // 128, 128), jnp.float32),
            pltpu.VMEM((B // 128, 128), jnp.float32),
            pltpu.VMEM((B // 128, 128), jnp.float32),
            pltpu.VMEM((B // 128, 128), jnp.float32),
            pltpu.SemaphoreType.DMA,
            pltpu.SemaphoreType.DMA,
        ],
        compiler_params=pltpu.CompilerParams(needs_layout_passes=False),
    )
    def k(ub_hbm, lb_hbm, lbl_hbm, out_ub, out_lb,
          lbl_v, ubr, lbr, oub0, oub1, olb0, olb1, sem_in, sem_out):
        wid = lax.axis_index("s") * NCORES + lax.axis_index("c")
        start = lax.shift_right_logical(wid * 125, 2)
        flat_start = start * NCLS
        pltpu.sync_copy(lbl_hbm, lbl_v)
        cin_u = pltpu.async_copy(
            ub_hbm.at[pl.ds(flat_start, WROWS * NCLS)], ubr, sem_in)
        cin_l = pltpu.async_copy(
            lb_hbm.at[pl.ds(flat_start, WROWS * NCLS)], lbr, sem_in)
        cin_u.wait()
        cin_l.wait()

        obufs = ((oub0, olb0), (oub1, olb1))
        pending = []
        nsub = B // 128
        for r in range(WROWS):
            ob_u, ob_l = obufs[r % 2]
            if r >= 2:
                pending[2 * (r - 2)].wait()
                pending[2 * (r - 2) + 1].wait()
            row_base = jnp.full((LANES,), r * NCLS, jnp.int32)

            @plsc.parallel_loop(0, nsub)
            def sbody(s, ob_u=ob_u, ob_l=ob_l, row_base=row_base):
                soff = pl.multiple_of(s * 128, 128)
                for kk in range(8):
                    idx = lbl_v[pl.ds(soff + kk * LANES, LANES)] + row_base
                    ob_u[s, pl.ds(kk * LANES, LANES)] = plsc.load_gather(
                        ubr, [idx])
                    ob_l[s, pl.ds(kk * LANES, LANES)] = plsc.load_gather(
                        lbr, [idx])
            pending.append(pltpu.async_copy(ob_u, out_ub.at[start + r], sem_out))
            pending.append(pltpu.async_copy(ob_l, out_lb.at[start + r], sem_out))
        for cp in pending[2 * (WROWS - 2):]:
            cp.wait()

    return k(ub_flat, lb_flat, labels)


def _tc_loss(labels, f3, ub3, lb3):
    n_rows = f3.shape[0]
    block_rows = 256
    grid = (n_rows // block_rows,)

    def body(lbl_ref, f_ref, ub_hbm, lb_hbm, out_ref, ub_v, lb_v, acc_ref, sem):
        i = pl.program_id(0)

        @pl.when(i == 0)
        def _():
            cu = pltpu.make_async_copy(ub_hbm, ub_v, sem)
            cu.start()
            cu.wait()
            cl = pltpu.make_async_copy(lb_hbm, lb_v, sem)
            cl.start()
            cl.wait()
            acc_ref[...] = jnp.zeros_like(acc_ref)

        base = i * block_rows

        def grp(g, acc):
            r0 = g * 4
            terms = []
            for k in range(4):
                l = lbl_ref[base + r0 + k]
                f = f_ref[r0 + k]
                terms.append(jnp.maximum(lb_v[l] - f, 0.0)
                             + jnp.maximum(f - ub_v[l], 0.0))
            return acc + ((terms[0] + terms[1]) + (terms[2] + terms[3]))

        acc_ref[...] = lax.fori_loop(0, block_rows // 4, grp, acc_ref[...])

        @pl.when(i == grid[0] - 1)
        def _():
            out_ref[0] = jnp.sum(acc_ref[...]) / (float(n_rows) * float(n_rows))

    grid_spec = pltpu.PrefetchScalarGridSpec(
        num_scalar_prefetch=1,
        grid=grid,
        in_specs=[
            pl.BlockSpec((block_rows, 32, 128), lambda i, lbl: (i, 0, 0)),
            pl.BlockSpec(memory_space=pltpu.MemorySpace.HBM),
            pl.BlockSpec(memory_space=pltpu.MemorySpace.HBM),
        ],
        out_specs=pl.BlockSpec(memory_space=pltpu.MemorySpace.SMEM),
        scratch_shapes=[
            pltpu.VMEM((NCLS, 32, 128), jnp.float32),
            pltpu.VMEM((NCLS, 32, 128), jnp.float32),
            pltpu.VMEM((32, 128), jnp.float32),
            pltpu.SemaphoreType.DMA,
        ],
    )
    return pl.pallas_call(
        body,
        grid_spec=grid_spec,
        out_shape=jax.ShapeDtypeStruct((1,), jnp.float32),
    )(labels, f3, ub3, lb3)


def kernel(FSR_Mat, labels, class_smi_UB, class_smi_LB):
    t_ub, t_lb = _sc_gather_cols(
        class_smi_UB.reshape(-1), class_smi_LB.reshape(-1), labels)
    b = FSR_Mat.shape[0]
    f3 = FSR_Mat.reshape(b, 32, 128)
    return _tc_loss(labels, f3, t_ub, t_lb)[0]
